# baseline (speedup 1.0000x reference)
import functools
import math

import jax
import jax.numpy as jnp
from jax import lax
from jax.experimental import pallas as pl
from jax.experimental.pallas import tpu as pltpu
from jax.experimental.pallas import tpu_sc as plsc

_D = 64
_LANES = 16
_NC = 2
_NS = 16
_NW = _NC * _NS
_G = 128
_PITCH = _G
_RPITCH = 2 * _D + 1
_CHUNK = 128
_CPB = _G // _CHUNK
_SB = 4
_SCALE = math.sqrt(_D)


@functools.lru_cache(maxsize=None)
def _build(n_j: int, n_b: int):
    n_blk = n_b // _G
    n_tasks = n_j * n_blk
    assert n_tasks % (_NW * _SB) == 0
    tpw = n_tasks // _NW

    mesh = plsc.VectorSubcoreMesh(
        core_axis_name="c", subcore_axis_name="s",
        num_cores=_NC, num_subcores=_NS)

    @functools.partial(
        pl.kernel,
        out_type=jax.ShapeDtypeStruct((n_j, _D, n_b), jnp.float32),
        mesh=mesh,
        compiler_params=pltpu.CompilerParams(needs_layout_passes=False),
        scratch_types=[
            pltpu.VMEM((2, _SB * _CPB, _CHUNK), jnp.int32),
            pltpu.VMEM((2, _SB * _G), jnp.int32),
            pltpu.VMEM((2, _G, _RPITCH), jnp.float32),
            pltpu.VMEM((2, _D, _PITCH), jnp.float32),
            pltpu.SemaphoreType.DMA((2,)),
            pltpu.SemaphoreType.DMA((2,)),
            pltpu.SemaphoreType.DMA((2,)),
        ],
    )
    def sc_embed(idx_hbm, table_hbm, out_hbm, idx_v, poff_v,
                 rows_v, tr_v, isem, gsem, wsem):
        wid = lax.axis_index("s") * _NC + lax.axis_index("c")
        t0 = wid * tpw
        iot = lax.iota(jnp.int32, _LANES)

        def idx_src(sb):
            g = t0 + sb * _SB
            j = g // n_blk
            h = (g % n_blk) // _SB
            return idx_hbm.at[j, pl.ds(h * (_SB * _CPB), _SB * _CPB)]

        def out_dst(t):
            g = t0 + t
            j = g // n_blk
            blk = g % n_blk
            return out_hbm.at[j, :, pl.ds(blk * _G, _G)]

        def fire_idx(sb):
            si = sb % 2
            pltpu.async_copy(idx_src(sb), idx_v.at[si], isem.at[si])

        def wait_idx(sb):
            si = sb % 2
            pltpu.make_async_copy(idx_src(sb), idx_v.at[si],
                                  isem.at[si]).wait()

        def compute_halves(sb):
            si = sb % 2

            def m_body(m, carry):
                r = m // (_CHUNK // _LANES)
                cc = (m % (_CHUNK // _LANES)) * _LANES
                v = idx_v[si, r, pl.ds(cc, _LANES)]
                poff_v[si, pl.ds(m * _LANES, _LANES)] = lax.shift_left(
                    lax.bitwise_and(v, 1), 6)
                idx_v[si, r, pl.ds(cc, _LANES)] = lax.shift_right_logical(
                    v, 1)
                return carry

            lax.fori_loop(0, (_SB * _G) // _LANES, m_body, 0, unroll=8)

        def fire_gather(t):
            p = t % 2
            si = (t // _SB) % 2
            for c in range(_CPB):
                pltpu.async_copy(
                    table_hbm.at[idx_v.at[si, (t % _SB) * _CPB + c]],
                    rows_v.at[p, pl.ds(c * _CHUNK, _CHUNK),
                              pl.ds(0, 2 * _D)], gsem.at[p])

        def wait_gather(t):
            p = t % 2
            si = (t // _SB) % 2
            for c in range(_CPB):
                pltpu.make_async_copy(
                    table_hbm.at[idx_v.at[si, (t % _SB) * _CPB + c]],
                    rows_v.at[p, pl.ds(c * _CHUNK, _CHUNK),
                              pl.ds(0, 2 * _D)],
                    gsem.at[p]).wait()

        def fire_write(t):
            p = t % 2
            pltpu.async_copy(tr_v.at[p, :, pl.ds(0, _G)], out_dst(t),
                             wsem.at[p])

        def wait_write(t):
            p = t % 2
            pltpu.make_async_copy(tr_v.at[p, :, pl.ds(0, _G)], out_dst(t),
                                  wsem.at[p]).wait()

        fire_idx(0)
        wait_idx(0)
        compute_halves(0)
        fire_gather(0)

        def task_body(t, carry):
            p = t % 2
            si = (t // _SB) % 2

            @pl.when(jnp.logical_and(t % _SB == _SB - 1, t + 1 < tpw))
            def _():
                wait_idx(t // _SB + 1)
                compute_halves(t // _SB + 1)

            @pl.when(t + 1 < tpw)
            def _():
                fire_gather(t + 1)

            wait_gather(t)

            @pl.when(jnp.logical_and(t % _SB == 0, t + _SB < tpw))
            def _():
                fire_idx(t // _SB + 1)

            @pl.when(t >= 2)
            def _():
                wait_write(t - 2)

            def tr_body(b16, carry2):
                b0 = b16 * _LANES
                bvec = iot + b0
                po = poff_v[si, pl.ds((t % _SB) * _G + b0, _LANES)]
                for d in range(_D):
                    v = plsc.load_gather(rows_v.at[p], [bvec, po + d])
                    tr_v[p, d, pl.ds(b0, _LANES)] = v * _SCALE
                return carry2

            lax.fori_loop(0, _G // _LANES, tr_body, 0)
            fire_write(t)
            return carry

        lax.fori_loop(0, tpw, task_body, 0)
        wait_write(tpw - 2)
        wait_write(tpw - 1)

    return sc_embed


def kernel(x, table):
    n_b, n_j = x.shape
    idx3 = x.T.reshape(n_j, n_b // _CHUNK, _CHUNK)
    tbl2 = table.reshape(table.shape[0] // 2, 2 * _D)
    out = _build(n_j, n_b)(idx3, tbl2)
    return out.transpose(2, 0, 1)


# speedup vs baseline: 1.5728x; 1.5728x over previous
import functools
import math

import jax
import jax.numpy as jnp
from jax import lax
from jax.experimental import pallas as pl
from jax.experimental.pallas import tpu as pltpu
from jax.experimental.pallas import tpu_sc as plsc

_D = 64
_LANES = 16
_NC = 2
_NS = 16
_NW = _NC * _NS
_B = 8
_CHUNK = 128
_SCALE = math.sqrt(_D)


@functools.lru_cache(maxsize=None)
def _build(n_j: int, n_b: int):
    npt = _B * n_j
    n_tasks = n_b // _B
    assert n_tasks % _NW == 0
    tpw = n_tasks // _NW
    chunks = []
    off = 0
    while off < npt:
        c = min(_CHUNK, npt - off)
        chunks.append((off, c))
        off += c

    mesh = plsc.VectorSubcoreMesh(
        core_axis_name="c", subcore_axis_name="s",
        num_cores=_NC, num_subcores=_NS)

    @functools.partial(
        pl.kernel,
        out_type=jax.ShapeDtypeStruct((n_tasks, npt, _D), jnp.float32),
        mesh=mesh,
        compiler_params=pltpu.CompilerParams(
            use_tc_tiling_on_sc=False, needs_layout_passes=False),
        scratch_types=[
            pltpu.VMEM((2, npt), jnp.int32),
            pltpu.VMEM((2, npt, _D), jnp.float32),
            pltpu.SemaphoreType.DMA((2,)),
            pltpu.SemaphoreType.DMA((2,)),
            pltpu.SemaphoreType.DMA((2,)),
        ],
    )
    def sc_embed(idx_hbm, table_hbm, out_hbm, idx_v, rows_v,
                 isem, gsem, wsem):
        wid = lax.axis_index("s") * _NC + lax.axis_index("c")
        t0 = wid * tpw

        def fire_idx(t, p):
            pltpu.async_copy(idx_hbm.at[t0 + t], idx_v.at[p], isem.at[p])

        def wait_idx(t, p):
            pltpu.make_async_copy(idx_hbm.at[t0 + t], idx_v.at[p],
                                  isem.at[p]).wait()

        def fire_gather(t, p):
            for off, c in chunks:
                pltpu.async_copy(
                    table_hbm.at[idx_v.at[p, pl.ds(off, c)]],
                    rows_v.at[p, pl.ds(off, c)], gsem.at[p])

        def wait_gather(t, p):
            for off, c in chunks:
                pltpu.make_async_copy(
                    table_hbm.at[idx_v.at[p, pl.ds(off, c)]],
                    rows_v.at[p, pl.ds(off, c)], gsem.at[p]).wait()

        def fire_write(t, p):
            pltpu.async_copy(rows_v.at[p], out_hbm.at[t0 + t], wsem.at[p])

        def wait_write(t, p):
            pltpu.make_async_copy(rows_v.at[p], out_hbm.at[t0 + t],
                                  wsem.at[p]).wait()

        fire_idx(0, 0)
        fire_idx(1, 1)
        wait_idx(0, 0)
        fire_gather(0, 0)

        def pair_body(t2, carry):
            for p in range(2):
                t = t2 * 2 + p
                q = 1 - p
                @pl.when(t + 1 < tpw)
                def _():
                    wait_idx(t + 1, q)

                    @pl.when(t >= 1)
                    def _():
                        wait_write(t - 1, q)

                    fire_gather(t + 1, q)

                wait_gather(t, p)

                @pl.when(t + 2 < tpw)
                def _():
                    fire_idx(t + 2, p)

                def sc_body(m, carry2):
                    r = m // (_D // _LANES)
                    cc = (m % (_D // _LANES)) * _LANES
                    rows_v[p, r, pl.ds(cc, _LANES)] = (
                        rows_v[p, r, pl.ds(cc, _LANES)] * _SCALE)
                    return carry2

                lax.fori_loop(0, (npt * _D) // _LANES, sc_body, 0,
                              unroll=8)
                fire_write(t, p)
            return carry

        lax.fori_loop(0, tpw // 2, pair_body, 0)
        wait_write(tpw - 2, 0)
        wait_write(tpw - 1, 1)

    return sc_embed


def kernel(x, table):
    n_b, n_j = x.shape
    idx2 = x.reshape(n_b // _B, _B * n_j)
    out = _build(n_j, n_b)(idx2, table)
    return out.reshape(n_b, n_j, _D)
